# Initial kernel scaffold; baseline (speedup 1.0000x reference)
import jax
import jax.numpy as jnp
from jax.experimental import pallas as pl


def kernel(atom_element, atom_charge, atom_aromatic, atom_hybridization, atom_in_ring, edge_index, edge_attr, params):
    raise NotImplementedError("write your pallas kernel here")


# speedup vs baseline: 2.6405x; 2.6405x over previous
import jax
import jax.numpy as jnp
from jax import lax
from jax.experimental import pallas as pl
from jax.experimental.pallas import tpu as pltpu
from jax.experimental.pallas import tpu_sc as plsc

_N = 10000
_E = 320000
_H = 128
_NPAD = 10240
_BLK_N = 2048
_BLK_E = 512
_NC = 2
_NS = 16
_NW = _NC * _NS
_EPW = _E // _NW
_EPS = _E // _NS
_CH = 80
_ZR = 16
_RPS = _NPAD // _NS

_f32 = jnp.float32


def _enc_node_body(elem_ref, charge_ref, arom_ref, hyb_ref, ring_ref,
                   te_ref, tc_ref, ta_ref, th_ref, tr_ref, bias_ref, out_ref):
    blk = elem_ref.shape[0]

    def oh(iref, k):
        ids = iref[:]
        return (ids[:, None] ==
                lax.broadcasted_iota(jnp.int32, (blk, k), 1)).astype(_f32)

    acc = oh(elem_ref, 16) @ te_ref[:]
    acc += charge_ref[:].astype(_f32)[:, None] * tc_ref[:]
    acc += oh(arom_ref, 8) @ ta_ref[:]
    acc += oh(hyb_ref, 8) @ th_ref[:]
    acc += oh(ring_ref, 8) @ tr_ref[:]
    out_ref[:] = acc + bias_ref[:]


def _enc_edge_body(bt_ref, bc_ref, br_ref, tb_ref, tc_ref, tr_ref, bias_ref,
                   out_ref):
    blk = bt_ref.shape[0]

    def oh(iref):
        ids = iref[:]
        return (ids[:, None] ==
                lax.broadcasted_iota(jnp.int32, (blk, 8), 1)).astype(_f32)

    out_ref[:] = (oh(bt_ref) @ tb_ref[:] + oh(bc_ref) @ tc_ref[:]
                  + oh(br_ref) @ tr_ref[:] + bias_ref[:])


def _node_mm_body(h_ref, wbd_ref, bbd_ref, we_ref, be_ref, tbd_ref, te_ref):
    h = h_ref[:]
    tbd_ref[:] = h @ wbd_ref[:] + bbd_ref[:]
    te_ref[:] = h @ we_ref[:] + be_ref[:]


def _edge_fuse_body(e_ref, gbd_ref, ge_ref, wc_ref, bc_ref, eout_ref, pq_ref):
    e = e_ref[:]
    ehat = e @ wc_ref[:] + bc_ref[:] + gbd_ref[:, _H:] + ge_ref[:]
    sig = jax.nn.sigmoid(ehat)
    eout_ref[:] = e + jnp.maximum(ehat, 0.0)
    pq_ref[0] = sig * gbd_ref[:, :_H]
    pq_ref[1] = sig


def _h_update_body(h_ref, nd_ref, wa_ref, ba_ref, out_ref):
    h = h_ref[:]
    upd = h @ wa_ref[:] + ba_ref[:] + nd_ref[0] / (nd_ref[1] + 1e-6)
    out_ref[:] = h + jnp.maximum(upd, 0.0)


def _full(shape):
    return pl.BlockSpec(shape, lambda i: (0,) * len(shape))


def _enc_node(elem, charge, arom, hyb, ring, te, tc, ta, th, tr, bias):
    ids = pl.BlockSpec((_BLK_N,), lambda i: (i,))
    return pl.pallas_call(
        _enc_node_body,
        grid=(_NPAD // _BLK_N,),
        in_specs=[ids, ids, ids, ids, ids, _full((16, _H)), _full((1, _H)),
                  _full((8, _H)), _full((8, _H)), _full((8, _H)),
                  _full((1, _H))],
        out_specs=pl.BlockSpec((_BLK_N, _H), lambda i: (i, 0)),
        out_shape=jax.ShapeDtypeStruct((_NPAD, _H), _f32),
    )(elem, charge, arom, hyb, ring, te, tc, ta, th, tr, bias)


def _enc_edge(bt, bc, br, tb, tc, tr, bias):
    ids = pl.BlockSpec((_BLK_E,), lambda i: (i,))
    return pl.pallas_call(
        _enc_edge_body,
        grid=(_E // _BLK_E,),
        in_specs=[ids, ids, ids, _full((8, _H)), _full((8, _H)),
                  _full((8, _H)), _full((1, _H))],
        out_specs=pl.BlockSpec((_BLK_E, _H), lambda i: (i, 0)),
        out_shape=jax.ShapeDtypeStruct((_E, _H), _f32),
    )(bt, bc, br, tb, tc, tr, bias)


def _node_mm(h, wbd, bbd, we, be):
    return pl.pallas_call(
        _node_mm_body,
        grid=(_NPAD // _BLK_N,),
        in_specs=[pl.BlockSpec((_BLK_N, _H), lambda i: (i, 0)),
                  _full((_H, 2 * _H)), _full((1, 2 * _H)),
                  _full((_H, _H)), _full((1, _H))],
        out_specs=[pl.BlockSpec((_BLK_N, 2 * _H), lambda i: (i, 0)),
                   pl.BlockSpec((_BLK_N, _H), lambda i: (i, 0))],
        out_shape=[jax.ShapeDtypeStruct((_NPAD, 2 * _H), _f32),
                   jax.ShapeDtypeStruct((_NPAD, _H), _f32)],
    )(h, wbd, bbd, we, be)


def _edge_fuse(e, gbd, ge, wc, bc):
    return pl.pallas_call(
        _edge_fuse_body,
        grid=(_E // _BLK_E,),
        in_specs=[pl.BlockSpec((_BLK_E, _H), lambda i: (i, 0)),
                  pl.BlockSpec((_BLK_E, 2 * _H), lambda i: (i, 0)),
                  pl.BlockSpec((_BLK_E, _H), lambda i: (i, 0)),
                  _full((_H, _H)), _full((1, _H))],
        out_specs=[pl.BlockSpec((_BLK_E, _H), lambda i: (i, 0)),
                   pl.BlockSpec((2, _BLK_E, _H), lambda i: (0, i, 0))],
        out_shape=[jax.ShapeDtypeStruct((_E, _H), _f32),
                   jax.ShapeDtypeStruct((2, _E, _H), _f32)],
    )(e, gbd, ge, wc, bc)


def _h_update(h, nd, wa, ba):
    return pl.pallas_call(
        _h_update_body,
        grid=(_NPAD // _BLK_N,),
        in_specs=[pl.BlockSpec((_BLK_N, _H), lambda i: (i, 0)),
                  pl.BlockSpec((2, _BLK_N, _H), lambda i: (0, i, 0)),
                  _full((_H, _H)), _full((1, _H))],
        out_specs=pl.BlockSpec((_BLK_N, _H), lambda i: (i, 0)),
        out_shape=jax.ShapeDtypeStruct((_NPAD, _H), _f32),
    )(h, nd, wa, ba)


def _sc_gather_body(src_hbm, dst_hbm, tbd_hbm, te_hbm, gbd_hbm, ge_hbm,
                    idx_s, idx_d, rows_bd, rows_e, sem1, sem2):
    wid = lax.axis_index("c") * _NS + lax.axis_index("s")
    base0 = wid * _EPW

    def chunk(i, carry):
        base = base0 + i * _CH
        pltpu.sync_copy(src_hbm.at[pl.ds(base, _CH)], idx_s)
        pltpu.sync_copy(dst_hbm.at[pl.ds(base, _CH)], idx_d)
        cp1 = pltpu.async_copy(tbd_hbm.at[idx_s], rows_bd, sem1)
        cp2 = pltpu.async_copy(te_hbm.at[idx_d], rows_e, sem2)
        cp1.wait()
        cp2.wait()
        pltpu.sync_copy(rows_bd, gbd_hbm.at[pl.ds(base, _CH)])
        pltpu.sync_copy(rows_e, ge_hbm.at[pl.ds(base, _CH)])
        return carry

    lax.fori_loop(0, _EPW // _CH, chunk, 0)


def _sc_scatter_body(pq_hbm, dst_hbm, nd_hbm, acc, idx, rows, zbuf):
    c = lax.axis_index("c")
    s = lax.axis_index("s")

    for i in range(_ZR):
        for j in range(_H // 16):
            zbuf[i, pl.ds(j * 16, 16)] = jnp.zeros((16,), _f32)

    def zero_chunk(k, carry):
        pltpu.sync_copy(zbuf, acc.at[pl.ds(s * _RPS + k * _ZR, _ZR)])
        return carry

    lax.fori_loop(0, _RPS // _ZR, zero_chunk, 0)
    plsc.subcore_barrier()

    base0 = s * _EPS

    def chunk(i, carry):
        base = base0 + i * _CH
        pltpu.sync_copy(dst_hbm.at[pl.ds(base, _CH)], idx)
        pltpu.sync_copy(pq_hbm.at[c, pl.ds(base, _CH)], rows)
        pltpu.sync_copy(rows, acc.at[idx], add=True)
        return carry

    lax.fori_loop(0, _EPS // _CH, chunk, 0)
    plsc.subcore_barrier()

    pltpu.sync_copy(acc.at[pl.ds(s * _RPS, _RPS)],
                    nd_hbm.at[c, pl.ds(s * _RPS, _RPS)])


def _sc_gather(src, dst, tbd, te):
    mesh = plsc.VectorSubcoreMesh(core_axis_name="c", subcore_axis_name="s")
    return pl.kernel(
        _sc_gather_body,
        out_type=(jax.ShapeDtypeStruct((_E, 2 * _H), _f32),
                  jax.ShapeDtypeStruct((_E, _H), _f32)),
        mesh=mesh,
        scratch_types=[pltpu.VMEM((_CH,), jnp.int32),
                       pltpu.VMEM((_CH,), jnp.int32),
                       pltpu.VMEM((_CH, 2 * _H), _f32),
                       pltpu.VMEM((_CH, _H), _f32),
                       pltpu.SemaphoreType.DMA,
                       pltpu.SemaphoreType.DMA],
    )(src, dst, tbd, te)


def _sc_scatter(pq, dst):
    mesh = plsc.VectorSubcoreMesh(core_axis_name="c", subcore_axis_name="s")
    return pl.kernel(
        _sc_scatter_body,
        out_type=jax.ShapeDtypeStruct((2, _NPAD, _H), _f32),
        mesh=mesh,
        scratch_types=[pltpu.VMEM_SHARED((_NPAD, _H), _f32),
                       pltpu.VMEM((_CH,), jnp.int32),
                       pltpu.VMEM((_CH, _H), _f32),
                       pltpu.VMEM((_ZR, _H), _f32)],
    )(pq, dst)


def kernel(atom_element, atom_charge, atom_aromatic, atom_hybridization,
           atom_in_ring, edge_index, edge_attr, params):
    w_atom = params["atom_proj"]["w"]
    te = jnp.zeros((16, _H), _f32).at[:13].set(params["elem_emb"] @ w_atom[0:32])
    tc = params["charge_proj"]["w"] @ w_atom[32:40]
    bias_n = (params["charge_proj"]["b"] @ w_atom[32:40]
              + params["atom_proj"]["b"])[None, :]
    ta = jnp.zeros((8, _H), _f32).at[:2].set(params["aromatic_emb"] @ w_atom[40:48])
    th = jnp.zeros((8, _H), _f32).at[:6].set(params["hybrid_emb"] @ w_atom[48:64])
    tr = jnp.zeros((8, _H), _f32).at[:2].set(params["ring_emb"] @ w_atom[64:72])

    w_bond = params["bond_proj"]["w"]
    tb = jnp.zeros((8, _H), _f32).at[:5].set(params["bond_type_emb"] @ w_bond[0:16])
    tcj = jnp.zeros((8, _H), _f32).at[:2].set(params["bond_conj_emb"] @ w_bond[16:24])
    trg = jnp.zeros((8, _H), _f32).at[:2].set(params["bond_ring_emb"] @ w_bond[24:32])
    bias_e = params["bond_proj"]["b"][None, :]

    pad_n = _NPAD - _N
    elem = jnp.pad(atom_element.astype(jnp.int32), (0, pad_n))
    charge = jnp.pad(atom_charge.astype(jnp.int32), (0, pad_n))
    arom = jnp.pad(atom_aromatic.astype(jnp.int32), (0, pad_n))
    hyb = jnp.pad(atom_hybridization.astype(jnp.int32), (0, pad_n))
    ring = jnp.pad(atom_in_ring.astype(jnp.int32), (0, pad_n))
    src = edge_index[0].astype(jnp.int32)
    dst = edge_index[1].astype(jnp.int32)
    bt = edge_attr[:, 0].astype(jnp.int32)
    bc = edge_attr[:, 1].astype(jnp.int32)
    br = edge_attr[:, 2].astype(jnp.int32)

    h = _enc_node(elem, charge, arom, hyb, ring, te, tc, ta, th, tr, bias_n)
    e = _enc_edge(bt, bc, br, tb, tcj, trg, bias_e)

    for lp in params["layers"]:
        wbd = jnp.concatenate([lp["B"]["w"], lp["D"]["w"]], axis=1)
        bbd = jnp.concatenate([lp["B"]["b"], lp["D"]["b"]])[None, :]
        tbd, te_tab = _node_mm(h, wbd, bbd, lp["E"]["w"], lp["E"]["b"][None, :])
        gbd, ge = _sc_gather(src, dst, tbd, te_tab)
        e, pq = _edge_fuse(e, gbd, ge, lp["C"]["w"], lp["C"]["b"][None, :])
        nd = _sc_scatter(pq, dst)
        h = _h_update(h, nd, lp["A"]["w"], lp["A"]["b"][None, :])

    return h[:_N]


# speedup vs baseline: 3.3113x; 1.2540x over previous
import jax
import jax.numpy as jnp
from jax import lax
from jax.experimental import pallas as pl
from jax.experimental.pallas import tpu as pltpu
from jax.experimental.pallas import tpu_sc as plsc

_N = 10000
_E = 320000
_H = 128
_NPAD = 10240
_BLK_N = 2048
_BLK_E = 512
_NC = 2
_NS = 16
_NW = _NC * _NS
_EPW = _E // _NW
_EPS = _E // _NS
_CH = 80
_ZR = 16
_RPS = _NPAD // _NS

_f32 = jnp.float32


def _enc_node_body(elem_ref, charge_ref, arom_ref, hyb_ref, ring_ref,
                   te_ref, tc_ref, ta_ref, th_ref, tr_ref, bias_ref, out_ref):
    blk = elem_ref.shape[0]

    def oh(iref, k):
        ids = iref[:]
        return (ids[:, None] ==
                lax.broadcasted_iota(jnp.int32, (blk, k), 1)).astype(_f32)

    acc = oh(elem_ref, 16) @ te_ref[:]
    acc += charge_ref[:].astype(_f32)[:, None] * tc_ref[:]
    acc += oh(arom_ref, 8) @ ta_ref[:]
    acc += oh(hyb_ref, 8) @ th_ref[:]
    acc += oh(ring_ref, 8) @ tr_ref[:]
    out_ref[:] = acc + bias_ref[:]


def _enc_edge_body(bt_ref, bc_ref, br_ref, tb_ref, tc_ref, tr_ref, bias_ref,
                   out_ref):
    blk = bt_ref.shape[0]

    def oh(iref):
        ids = iref[:]
        return (ids[:, None] ==
                lax.broadcasted_iota(jnp.int32, (blk, 8), 1)).astype(_f32)

    out_ref[:] = (oh(bt_ref) @ tb_ref[:] + oh(bc_ref) @ tc_ref[:]
                  + oh(br_ref) @ tr_ref[:] + bias_ref[:])


def _node_mm_body(h_ref, wbd_ref, bbd_ref, we_ref, be_ref, tbd_ref, te_ref):
    h = h_ref[:]
    tbd_ref[:] = h @ wbd_ref[:] + bbd_ref[:]
    te_ref[:] = h @ we_ref[:] + be_ref[:]


def _edge_fuse_body(e_ref, gbd_ref, ge_ref, wc_ref, bc_ref, eout_ref, pq_ref):
    e = e_ref[:]
    ehat = e @ wc_ref[:] + bc_ref[:] + gbd_ref[:, _H:] + ge_ref[:]
    sig = jax.nn.sigmoid(ehat)
    eout_ref[:] = e + jnp.maximum(ehat, 0.0)
    pq_ref[0] = sig * gbd_ref[:, :_H]
    pq_ref[1] = sig


def _h_update_body(h_ref, nd_ref, wa_ref, ba_ref, out_ref):
    h = h_ref[:]
    upd = h @ wa_ref[:] + ba_ref[:] + nd_ref[0] / (nd_ref[1] + 1e-6)
    out_ref[:] = h + jnp.maximum(upd, 0.0)


def _full(shape):
    return pl.BlockSpec(shape, lambda i: (0,) * len(shape))


def _enc_node(elem, charge, arom, hyb, ring, te, tc, ta, th, tr, bias):
    ids = pl.BlockSpec((_BLK_N,), lambda i: (i,))
    return pl.pallas_call(
        _enc_node_body,
        grid=(_NPAD // _BLK_N,),
        in_specs=[ids, ids, ids, ids, ids, _full((16, _H)), _full((1, _H)),
                  _full((8, _H)), _full((8, _H)), _full((8, _H)),
                  _full((1, _H))],
        out_specs=pl.BlockSpec((_BLK_N, _H), lambda i: (i, 0)),
        out_shape=jax.ShapeDtypeStruct((_NPAD, _H), _f32),
    )(elem, charge, arom, hyb, ring, te, tc, ta, th, tr, bias)


def _enc_edge(bt, bc, br, tb, tc, tr, bias):
    ids = pl.BlockSpec((_BLK_E,), lambda i: (i,))
    return pl.pallas_call(
        _enc_edge_body,
        grid=(_E // _BLK_E,),
        in_specs=[ids, ids, ids, _full((8, _H)), _full((8, _H)),
                  _full((8, _H)), _full((1, _H))],
        out_specs=pl.BlockSpec((_BLK_E, _H), lambda i: (i, 0)),
        out_shape=jax.ShapeDtypeStruct((_E, _H), _f32),
    )(bt, bc, br, tb, tc, tr, bias)


def _node_mm(h, wbd, bbd, we, be):
    return pl.pallas_call(
        _node_mm_body,
        grid=(_NPAD // _BLK_N,),
        in_specs=[pl.BlockSpec((_BLK_N, _H), lambda i: (i, 0)),
                  _full((_H, 2 * _H)), _full((1, 2 * _H)),
                  _full((_H, _H)), _full((1, _H))],
        out_specs=[pl.BlockSpec((_BLK_N, 2 * _H), lambda i: (i, 0)),
                   pl.BlockSpec((_BLK_N, _H), lambda i: (i, 0))],
        out_shape=[jax.ShapeDtypeStruct((_NPAD, 2 * _H), _f32),
                   jax.ShapeDtypeStruct((_NPAD, _H), _f32)],
    )(h, wbd, bbd, we, be)


def _edge_fuse(e, gbd, ge, wc, bc):
    return pl.pallas_call(
        _edge_fuse_body,
        grid=(_E // _BLK_E,),
        in_specs=[pl.BlockSpec((_BLK_E, _H), lambda i: (i, 0)),
                  pl.BlockSpec((_BLK_E, 2 * _H), lambda i: (i, 0)),
                  pl.BlockSpec((_BLK_E, _H), lambda i: (i, 0)),
                  _full((_H, _H)), _full((1, _H))],
        out_specs=[pl.BlockSpec((_BLK_E, _H), lambda i: (i, 0)),
                   pl.BlockSpec((2, _BLK_E, _H), lambda i: (0, i, 0))],
        out_shape=[jax.ShapeDtypeStruct((_E, _H), _f32),
                   jax.ShapeDtypeStruct((2, _E, _H), _f32)],
    )(e, gbd, ge, wc, bc)


def _h_update(h, nd, wa, ba):
    return pl.pallas_call(
        _h_update_body,
        grid=(_NPAD // _BLK_N,),
        in_specs=[pl.BlockSpec((_BLK_N, _H), lambda i: (i, 0)),
                  pl.BlockSpec((2, _BLK_N, _H), lambda i: (0, i, 0)),
                  _full((_H, _H)), _full((1, _H))],
        out_specs=pl.BlockSpec((_BLK_N, _H), lambda i: (i, 0)),
        out_shape=jax.ShapeDtypeStruct((_NPAD, _H), _f32),
    )(h, nd, wa, ba)


def _sc_gather_body(src_hbm, dst_hbm, tbd_hbm, te_hbm, gbd_hbm, ge_hbm,
                    idx_s, idx_d, rbd, re,
                    gb0, ge0, gb1, ge1, wb0, we0, wb1, we1):
    wid = lax.axis_index("c") * _NS + lax.axis_index("s")
    base0 = wid * _EPW
    pltpu.sync_copy(src_hbm.at[pl.ds(base0, _EPW)], idx_s)
    pltpu.sync_copy(dst_hbm.at[pl.ds(base0, _EPW)], idx_d)
    sems_g = (gb0, ge0, gb1, ge1)
    sems_w = (wb0, we0, wb1, we1)

    def g_start(ci, b):
        pltpu.async_copy(tbd_hbm.at[idx_s.at[pl.ds(ci * _CH, _CH)]],
                         rbd.at[b], sems_g[2 * b])
        pltpu.async_copy(te_hbm.at[idx_d.at[pl.ds(ci * _CH, _CH)]],
                         re.at[b], sems_g[2 * b + 1])

    def g_wait(b):
        pltpu.make_async_copy(tbd_hbm.at[idx_s.at[pl.ds(0, _CH)]],
                              rbd.at[b], sems_g[2 * b]).wait()
        pltpu.make_async_copy(te_hbm.at[idx_d.at[pl.ds(0, _CH)]],
                              re.at[b], sems_g[2 * b + 1]).wait()

    def w_start(ci, b):
        base = base0 + ci * _CH
        pltpu.async_copy(rbd.at[b], gbd_hbm.at[pl.ds(base, _CH)],
                         sems_w[2 * b])
        pltpu.async_copy(re.at[b], ge_hbm.at[pl.ds(base, _CH)],
                         sems_w[2 * b + 1])

    def w_wait(b):
        pltpu.make_async_copy(rbd.at[b], gbd_hbm.at[pl.ds(base0, _CH)],
                              sems_w[2 * b]).wait()
        pltpu.make_async_copy(re.at[b], ge_hbm.at[pl.ds(base0, _CH)],
                              sems_w[2 * b + 1]).wait()

    g_start(0, 0)

    def pair(j, carry):
        c0 = 2 * j
        g_wait(0)

        @pl.when(j > 0)
        def _():
            w_wait(1)

        g_start(c0 + 1, 1)
        w_start(c0, 0)
        g_wait(1)
        w_wait(0)
        g_start(c0 + 2, 0)
        w_start(c0 + 1, 1)
        return carry

    lax.fori_loop(0, (_EPW // _CH) // 2, pair, 0)
    last = _EPW // _CH - 1
    g_wait(0)
    w_wait(1)
    w_start(last, 0)
    w_wait(0)


def _sc_scatter_body(pq_hbm, dst_hbm, nd_hbm, acc, idx, rows, zbuf,
                     si0, sr0, ss0, si1, sr1, ss1):
    c = lax.axis_index("c")
    s = lax.axis_index("s")

    for i in range(_ZR):
        for j in range(_H // 16):
            zbuf[i, pl.ds(j * 16, 16)] = jnp.zeros((16,), _f32)

    def zero_chunk(k, carry):
        pltpu.sync_copy(zbuf, acc.at[pl.ds(s * _RPS + k * _ZR, _ZR)])
        return carry

    lax.fori_loop(0, _RPS // _ZR, zero_chunk, 0)
    plsc.subcore_barrier()

    base0 = s * _EPS
    sems_i = (si0, si1)
    sems_r = (sr0, sr1)
    sems_s = (ss0, ss1)

    def l_start(ci, b):
        base = base0 + ci * _CH
        pltpu.async_copy(dst_hbm.at[pl.ds(base, _CH)], idx.at[b], sems_i[b])
        pltpu.async_copy(pq_hbm.at[c, pl.ds(base, _CH)], rows.at[b],
                         sems_r[b])

    def l_wait(b):
        pltpu.make_async_copy(dst_hbm.at[pl.ds(base0, _CH)], idx.at[b],
                              sems_i[b]).wait()
        pltpu.make_async_copy(pq_hbm.at[c, pl.ds(base0, _CH)], rows.at[b],
                              sems_r[b]).wait()

    def s_start(b):
        pltpu.async_copy(rows.at[b], acc.at[idx.at[b]], sems_s[b], add=True)

    def s_wait(b):
        pltpu.make_async_copy(rows.at[b], acc.at[idx.at[b]],
                              sems_s[b]).wait()

    nch = _EPS // _CH
    l_start(0, 0)
    l_start(1, 1)

    def pair(j, carry):
        c0 = 2 * j
        l_wait(0)
        s_start(0)
        l_wait(1)
        s_start(1)
        s_wait(0)

        @pl.when(c0 + 2 < nch)
        def _():
            l_start(c0 + 2, 0)

        s_wait(1)

        @pl.when(c0 + 3 < nch)
        def _():
            l_start(c0 + 3, 1)

        return carry

    lax.fori_loop(0, nch // 2, pair, 0)
    plsc.subcore_barrier()

    pltpu.sync_copy(acc.at[pl.ds(s * _RPS, _RPS)],
                    nd_hbm.at[c, pl.ds(s * _RPS, _RPS)])


def _sc_gather(src, dst, tbd, te):
    mesh = plsc.VectorSubcoreMesh(core_axis_name="c", subcore_axis_name="s")
    return pl.kernel(
        _sc_gather_body,
        out_type=(jax.ShapeDtypeStruct((_E, 2 * _H), _f32),
                  jax.ShapeDtypeStruct((_E, _H), _f32)),
        mesh=mesh,
        scratch_types=[pltpu.VMEM((_EPW,), jnp.int32),
                       pltpu.VMEM((_EPW,), jnp.int32),
                       pltpu.VMEM((2, _CH, 2 * _H), _f32),
                       pltpu.VMEM((2, _CH, _H), _f32)]
                      + [pltpu.SemaphoreType.DMA] * 8,
    )(src, dst, tbd, te)


def _sc_scatter(pq, dst):
    mesh = plsc.VectorSubcoreMesh(core_axis_name="c", subcore_axis_name="s")
    return pl.kernel(
        _sc_scatter_body,
        out_type=jax.ShapeDtypeStruct((2, _NPAD, _H), _f32),
        mesh=mesh,
        scratch_types=[pltpu.VMEM_SHARED((_NPAD, _H), _f32),
                       pltpu.VMEM((2, _CH), jnp.int32),
                       pltpu.VMEM((2, _CH, _H), _f32),
                       pltpu.VMEM((_ZR, _H), _f32)]
                      + [pltpu.SemaphoreType.DMA] * 6,
    )(pq, dst)


def kernel(atom_element, atom_charge, atom_aromatic, atom_hybridization,
           atom_in_ring, edge_index, edge_attr, params):
    w_atom = params["atom_proj"]["w"]
    te = jnp.zeros((16, _H), _f32).at[:13].set(params["elem_emb"] @ w_atom[0:32])
    tc = params["charge_proj"]["w"] @ w_atom[32:40]
    bias_n = (params["charge_proj"]["b"] @ w_atom[32:40]
              + params["atom_proj"]["b"])[None, :]
    ta = jnp.zeros((8, _H), _f32).at[:2].set(params["aromatic_emb"] @ w_atom[40:48])
    th = jnp.zeros((8, _H), _f32).at[:6].set(params["hybrid_emb"] @ w_atom[48:64])
    tr = jnp.zeros((8, _H), _f32).at[:2].set(params["ring_emb"] @ w_atom[64:72])

    w_bond = params["bond_proj"]["w"]
    tb = jnp.zeros((8, _H), _f32).at[:5].set(params["bond_type_emb"] @ w_bond[0:16])
    tcj = jnp.zeros((8, _H), _f32).at[:2].set(params["bond_conj_emb"] @ w_bond[16:24])
    trg = jnp.zeros((8, _H), _f32).at[:2].set(params["bond_ring_emb"] @ w_bond[24:32])
    bias_e = params["bond_proj"]["b"][None, :]

    pad_n = _NPAD - _N
    elem = jnp.pad(atom_element.astype(jnp.int32), (0, pad_n))
    charge = jnp.pad(atom_charge.astype(jnp.int32), (0, pad_n))
    arom = jnp.pad(atom_aromatic.astype(jnp.int32), (0, pad_n))
    hyb = jnp.pad(atom_hybridization.astype(jnp.int32), (0, pad_n))
    ring = jnp.pad(atom_in_ring.astype(jnp.int32), (0, pad_n))
    src = edge_index[0].astype(jnp.int32)
    dst = edge_index[1].astype(jnp.int32)
    bt = edge_attr[:, 0].astype(jnp.int32)
    bc = edge_attr[:, 1].astype(jnp.int32)
    br = edge_attr[:, 2].astype(jnp.int32)

    h = _enc_node(elem, charge, arom, hyb, ring, te, tc, ta, th, tr, bias_n)
    e = _enc_edge(bt, bc, br, tb, tcj, trg, bias_e)

    for lp in params["layers"]:
        wbd = jnp.concatenate([lp["B"]["w"], lp["D"]["w"]], axis=1)
        bbd = jnp.concatenate([lp["B"]["b"], lp["D"]["b"]])[None, :]
        tbd, te_tab = _node_mm(h, wbd, bbd, lp["E"]["w"], lp["E"]["b"][None, :])
        gbd, ge = _sc_gather(src, dst, tbd, te_tab)
        e, pq = _edge_fuse(e, gbd, ge, lp["C"]["w"], lp["C"]["b"][None, :])
        nd = _sc_scatter(pq, dst)
        h = _h_update(h, nd, lp["A"]["w"], lp["A"]["b"][None, :])

    return h[:_N]
